# baseline (speedup 1.0000x reference)
import functools

import jax
import jax.numpy as jnp
from jax import lax
from jax.experimental import pallas as pl
from jax.experimental.pallas import tpu as pltpu
from jax.experimental.pallas import tpu_sc as plsc

NBUF = 8
INFLIGHT = 6
ROW_PAD = 56
COL_PAD = 128
IDX_PAD = 128


@functools.lru_cache(maxsize=None)
def _build(num_rows, row_len, dim):
    mesh = plsc.VectorSubcoreMesh(core_axis_name="c", subcore_axis_name="s")
    nc, ns = mesh.num_cores, mesh.num_subcores
    nw = nc * ns
    assert num_rows % nw == 0
    rows_per_w = num_rows // nw
    assert rows_per_w % NBUF == 0 and rows_per_w >= NBUF

    @functools.partial(
        pl.kernel,
        out_type=jax.ShapeDtypeStruct((num_rows, ROW_PAD, COL_PAD), jnp.float32),
        mesh=mesh,
        scratch_types=[
            pltpu.VMEM((rows_per_w, IDX_PAD), jnp.int32),
            pltpu.VMEM((NBUF, ROW_PAD, 64), jnp.float32),
        ]
        + [pltpu.SemaphoreType.DMA] * (2 * NBUF),
        compiler_params=pltpu.CompilerParams(use_tc_tiling_on_sc=False),
    )
    def emb(idx_hbm, table_hbm, out_hbm, idx_v, stag_v, *sems):
        gsems, ssems = sems[:NBUF], sems[NBUF:]
        wid = lax.axis_index("s") * nc + lax.axis_index("c")
        base = wid * rows_per_w
        pltpu.sync_copy(idx_hbm.at[pl.ds(base, rows_per_w)], idx_v)

        def gather(j, b):
            pltpu.async_copy(
                table_hbm.at[idx_v.at[j, pl.ds(0, ROW_PAD)]],
                stag_v.at[b],
                gsems[b],
            )

        def gather_wait(b):
            pltpu.make_async_copy(
                table_hbm.at[idx_v.at[0, pl.ds(0, ROW_PAD)]],
                stag_v.at[b],
                gsems[b],
            ).wait()

        def store_wait(b):
            pltpu.make_async_copy(
                stag_v.at[b],
                out_hbm.at[base, pl.ds(0, ROW_PAD), pl.ds(0, dim)],
                ssems[b],
            ).wait()

        for b in range(INFLIGHT):
            gather(b, b)

        @pl.loop(0, rows_per_w, step=NBUF)
        def _(g):
            for b in range(NBUF):
                j = g + b
                gather_wait(b)
                pltpu.async_copy(
                    stag_v.at[b],
                    out_hbm.at[base + j, pl.ds(0, ROW_PAD), pl.ds(0, dim)],
                    ssems[b],
                )
                nj = j + INFLIGHT
                sb = (b + INFLIGHT) % NBUF

                @pl.when(nj < rows_per_w)
                def _():
                    @pl.when(nj >= NBUF)
                    def _():
                        store_wait(sb)

                    gather(nj, sb)

        for b in range(NBUF):
            store_wait(b)

    return emb


def kernel(token_ids, weights):
    num_rows, row_len = token_ids.shape
    dim = weights.shape[1]
    emb = _build(num_rows, row_len, dim)
    idx_pad = jnp.pad(token_ids, ((0, 0), (0, IDX_PAD - row_len)))
    out56 = emb(idx_pad, weights)
    return lax.slice(out56, (0, 0, 0), (num_rows, row_len, dim))


# speedup vs baseline: 2.5080x; 2.5080x over previous
import functools

import jax
import jax.numpy as jnp
from jax import lax
from jax.experimental import pallas as pl
from jax.experimental.pallas import tpu as pltpu
from jax.experimental.pallas import tpu_sc as plsc

EMB_DIM = 64
CHUNK = 128
NBUF = 8
INFLIGHT = 6


@functools.lru_cache(maxsize=None)
def _build(num_flat, dim):
    mesh = plsc.VectorSubcoreMesh(core_axis_name="c", subcore_axis_name="s")
    nc, ns = mesh.num_cores, mesh.num_subcores
    nw = nc * ns
    assert num_flat % (nw * CHUNK) == 0
    nchunks = num_flat // (nw * CHUNK)
    assert nchunks % NBUF == 0 and nchunks >= NBUF

    @functools.partial(
        pl.kernel,
        out_type=jax.ShapeDtypeStruct((num_flat, dim), jnp.float32),
        mesh=mesh,
        scratch_types=[
            pltpu.VMEM((nchunks, CHUNK), jnp.int32),
            pltpu.VMEM((NBUF, CHUNK, dim), jnp.float32),
        ]
        + [pltpu.SemaphoreType.DMA] * (2 * NBUF),
        compiler_params=pltpu.CompilerParams(use_tc_tiling_on_sc=False),
    )
    def emb(idx_hbm, table_hbm, out_hbm, idx_v, rows_v, *sems):
        gsems, ssems = sems[:NBUF], sems[NBUF:]
        wid = lax.axis_index("s") * nc + lax.axis_index("c")
        base = wid * (nchunks * CHUNK)
        pltpu.sync_copy(idx_hbm.at[wid], idx_v)
        for b in range(INFLIGHT):
            pltpu.async_copy(table_hbm.at[idx_v.at[b]], rows_v.at[b], gsems[b])

        @pl.loop(0, nchunks, step=NBUF)
        def _(g):
            for b in range(NBUF):
                j = g + b
                pltpu.make_async_copy(
                    table_hbm.at[idx_v.at[b]], rows_v.at[b], gsems[b]
                ).wait()
                pltpu.async_copy(
                    rows_v.at[b], out_hbm.at[pl.ds(base + j * CHUNK, CHUNK)], ssems[b]
                )
                nj = j + INFLIGHT
                sb = (b + INFLIGHT) % NBUF

                @pl.when(nj < nchunks)
                def _():
                    @pl.when(nj >= NBUF)
                    def _():
                        pltpu.make_async_copy(
                            rows_v.at[sb],
                            out_hbm.at[pl.ds(base, CHUNK)],
                            ssems[sb],
                        ).wait()

                    pltpu.async_copy(
                        table_hbm.at[idx_v.at[nj]], rows_v.at[sb], gsems[sb]
                    )

        for b in range(NBUF):
            pltpu.make_async_copy(
                rows_v.at[b], out_hbm.at[pl.ds(base, CHUNK)], ssems[b]
            ).wait()

    return emb, nw, nchunks


def kernel(token_ids, weights):
    shape = token_ids.shape
    flat = token_ids.reshape(-1).astype(jnp.int32)
    emb, nw, nchunks = _build(flat.shape[0], weights.shape[1])
    idx3d = flat.reshape(nw, nchunks, CHUNK)
    out = emb(idx3d, weights)
    return out.reshape(*shape, weights.shape[1])
